# baseline (speedup 1.0000x reference)
import functools

import jax
import jax.numpy as jnp
from jax import lax
from jax.experimental import pallas as pl
from jax.experimental.pallas import tpu as pltpu
from jax.experimental.pallas import tpu_sc as plsc

NC = 2
NS = 16
NW = NC * NS
LANES = 16
GATHER_CHUNK = 128


def _make_sc_kernel(B, D, b_per_w):
    n_chunks = b_per_w // GATHER_CHUNK
    mesh = plsc.VectorSubcoreMesh(core_axis_name="c", subcore_axis_name="s")

    @functools.partial(
        pl.kernel,
        mesh=mesh,
        out_type=jax.ShapeDtypeStruct((NW, LANES), jnp.float32),
        compiler_params=pltpu.CompilerParams(use_tc_tiling_on_sc=False),
        scratch_types=[
            pltpu.VMEM((b_per_w,), jnp.int32),
            pltpu.VMEM((b_per_w, D), jnp.float32),
            pltpu.VMEM((b_per_w, D), jnp.float32),
            pltpu.VMEM((LANES,), jnp.float32),
            pltpu.SemaphoreType.DMA,
            pltpu.SemaphoreType.DMA,
        ],
    )
    def sc_kernel(emb_hbm, idx_hbm, tbl_hbm, out_hbm,
                  idx_v, emb_v, cent_v, res_v, sem_e, sem_g):
        wid = lax.axis_index("s") * NC + lax.axis_index("c")
        base = wid * b_per_w

        emb_cp = pltpu.async_copy(emb_hbm.at[pl.ds(base, b_per_w)], emb_v, sem_e)
        pltpu.sync_copy(idx_hbm.at[pl.ds(base, b_per_w)], idx_v)
        gathers = []
        for j in range(n_chunks):
            sl = pl.ds(j * GATHER_CHUNK, GATHER_CHUNK)
            gathers.append(
                pltpu.async_copy(tbl_hbm.at[idx_v.at[sl]], cent_v.at[sl], sem_g))
        emb_cp.wait()
        for g in gathers:
            g.wait()

        def body(r, accs):
            out = []
            for j in range(D // LANES):
                sl = pl.ds(j * LANES, LANES)
                d = emb_v[r, sl] - cent_v[r, sl]
                out.append(accs[j] + d * d)
            return tuple(out)

        zero = jnp.zeros((LANES,), jnp.float32)
        accs = lax.fori_loop(0, b_per_w, body, (zero,) * (D // LANES))
        total = accs[0]
        for a in accs[1:]:
            total = total + a
        res_v[...] = total
        pltpu.sync_copy(res_v, out_hbm.at[wid])

    return sc_kernel


def kernel(embedding_batch, label_batch, class_centers):
    B, D = embedding_batch.shape
    sc_kernel = _make_sc_kernel(B, D, B // NW)
    partials = sc_kernel(embedding_batch,
                         label_batch.astype(jnp.int32),
                         class_centers)
    return jnp.sum(partials) / B


# speedup vs baseline: 1.4850x; 1.4850x over previous
import functools

import jax
import jax.numpy as jnp
from jax import lax
from jax.experimental import pallas as pl
from jax.experimental.pallas import tpu as pltpu
from jax.experimental.pallas import tpu_sc as plsc

NC = 2
NS = 16
NW = NC * NS
LANES = 16


def _make_sc_kernel(B, D, b_per_w):
    n_chunks = b_per_w // LANES
    mesh = plsc.VectorSubcoreMesh(core_axis_name="c", subcore_axis_name="s")

    @functools.partial(
        pl.kernel,
        mesh=mesh,
        out_type=jax.ShapeDtypeStruct((NW, LANES), jnp.float32),
        scratch_types=[
            pltpu.VMEM((b_per_w,), jnp.int32),
            pltpu.VMEM((b_per_w, D), jnp.float32),
            pltpu.VMEM((LANES, 8, D), jnp.float32),
            pltpu.VMEM((LANES,), jnp.float32),
            pltpu.SemaphoreType.DMA,
            pltpu.SemaphoreType.DMA((LANES,)),
        ],
    )
    def sc_kernel(emb_hbm, idx_hbm, tbl_hbm, out_hbm,
                  idx_v, emb_v, blk_v, res_v, sem_e, sem_g):
        wid = lax.axis_index("s") * NC + lax.axis_index("c")
        base = wid * b_per_w

        emb_cp = pltpu.async_copy(emb_hbm.at[pl.ds(base, b_per_w)], emb_v, sem_e)
        pltpu.sync_copy(idx_hbm.at[pl.ds(base, b_per_w)], idx_v)

        def fire(l, slot):
            blk = pl.multiple_of((l >> 3) << 3, 8)
            pltpu.async_copy(tbl_hbm.at[pl.ds(blk, 8)], blk_v.at[slot],
                             sem_g.at[slot])

        v0 = idx_v[pl.ds(0, LANES)]
        for j in range(LANES):
            fire(v0[j], j)
        emb_cp.wait()

        def body(g, accs):
            out = list(accs)
            vc = idx_v[pl.ds(g * LANES, LANES)]
            gn = jnp.minimum(g + 1, n_chunks - 1)
            vn = idx_v[pl.ds(gn * LANES, LANES)]
            not_last = g + 1 < n_chunks
            for j in range(LANES):
                l = vc[j]
                sub = l & 7
                r = g * LANES + j
                pltpu.make_async_copy(
                    tbl_hbm.at[pl.ds(0, 8)], blk_v.at[j], sem_g.at[j]).wait()
                for f in range(D // LANES):
                    sl = pl.ds(f * LANES, LANES)
                    d = emb_v[r, sl] - blk_v[j, sub, sl]
                    out[f] = out[f] + d * d
                ln = vn[j]

                @pl.when(not_last)
                def _():
                    fire(ln, j)
            return tuple(out)

        zero = jnp.zeros((LANES,), jnp.float32)
        accs = lax.fori_loop(0, n_chunks, body, (zero,) * (D // LANES))
        total = accs[0]
        for a in accs[1:]:
            total = total + a
        res_v[...] = total
        pltpu.sync_copy(res_v, out_hbm.at[wid])

    return sc_kernel


def kernel(embedding_batch, label_batch, class_centers):
    B, D = embedding_batch.shape
    sc_kernel = _make_sc_kernel(B, D, B // NW)
    partials = sc_kernel(embedding_batch,
                         label_batch.astype(jnp.int32),
                         class_centers)
    return jnp.sum(partials) / B
